# Initial kernel scaffold; baseline (speedup 1.0000x reference)
import jax
import jax.numpy as jnp
from jax.experimental import pallas as pl


def kernel(x, z, edge_attr, max_action, params, edge_index):
    raise NotImplementedError("write your pallas kernel here")


# speedup vs baseline: 4.0724x; 4.0724x over previous
import functools

import jax
import jax.numpy as jnp
from jax import lax
from jax.experimental import pallas as pl
from jax.experimental.pallas import tpu as pltpu
from jax.experimental.pallas import tpu_sc as plsc

N = 10000
E = 320000
H = 128
HH = H // 2
MLP_H = 256
ACTION_DIM = 8

NB = 1000
EB = 2000
CH = 128
NCH = E // CH

def _vmesh():
    return plsc.VectorSubcoreMesh(core_axis_name="c", subcore_axis_name="s")


_NSUB = 16
_ZCHUNK = 1000
_NZ = N // _ZCHUNK


def _node1_body(hx_ref, win_ref, binn_ref, wall_ref, ball_ref,
                h_ref, ah_ref, db_ref, eh_ref):
    h = jnp.dot(hx_ref[...], win_ref[...],
                preferred_element_type=jnp.float32) + binn_ref[...]
    h_ref[...] = h
    allv = jnp.dot(h, wall_ref[...],
                   preferred_element_type=jnp.float32) + ball_ref[...]
    ah_ref[...] = allv[:, :H]
    db_ref[...] = allv[:, H:3 * H]
    eh_ref[...] = allv[:, 3 * H:]


def _node1(hx, win, binn, wall, ball):
    f32 = jnp.float32
    return pl.pallas_call(
        _node1_body,
        grid=(N // NB,),
        in_specs=[
            pl.BlockSpec((NB, H), lambda i: (i, 0)),
            pl.BlockSpec((H, H), lambda i: (0, 0)),
            pl.BlockSpec((1, H), lambda i: (0, 0)),
            pl.BlockSpec((H, 4 * H), lambda i: (0, 0)),
            pl.BlockSpec((1, 4 * H), lambda i: (0, 0)),
        ],
        out_specs=[
            pl.BlockSpec((NB, H), lambda i: (i, 0)),
            pl.BlockSpec((NB, H), lambda i: (i, 0)),
            pl.BlockSpec((NB, 2 * H), lambda i: (i, 0)),
            pl.BlockSpec((NB, H), lambda i: (i, 0)),
        ],
        out_shape=(
            jax.ShapeDtypeStruct((N, H), f32),
            jax.ShapeDtypeStruct((N, H), f32),
            jax.ShapeDtypeStruct((N, 2 * H), f32),
            jax.ShapeDtypeStruct((N, H), f32),
        ),
    )(hx, win, binn, wall, ball)


def _update_h(h_prev, ahp, nd):
    num = jnp.concatenate([nd[0, :, :HH], nd[1, :, :HH]], axis=1)
    den = jnp.concatenate([nd[0, :, HH:], nd[1, :, HH:]], axis=1) + 1e-6
    return h_prev + jnp.maximum(ahp + num / den, 0.0)


def _node2_body(h_ref, ahp_ref, nd_ref, wall_ref, ball_ref,
                h_out_ref, ah_ref, db_ref, eh_ref):
    h = _update_h(h_ref[...], ahp_ref[...], nd_ref[...])
    h_out_ref[...] = h
    allv = jnp.dot(h, wall_ref[...],
                   preferred_element_type=jnp.float32) + ball_ref[...]
    ah_ref[...] = allv[:, :H]
    db_ref[...] = allv[:, H:3 * H]
    eh_ref[...] = allv[:, 3 * H:]


def _node2(h_prev, ahp, nd, wall, ball):
    f32 = jnp.float32
    return pl.pallas_call(
        _node2_body,
        grid=(N // NB,),
        in_specs=[
            pl.BlockSpec((NB, H), lambda i: (i, 0)),
            pl.BlockSpec((NB, H), lambda i: (i, 0)),
            pl.BlockSpec((2, NB, H), lambda i: (0, i, 0)),
            pl.BlockSpec((H, 4 * H), lambda i: (0, 0)),
            pl.BlockSpec((1, 4 * H), lambda i: (0, 0)),
        ],
        out_specs=[
            pl.BlockSpec((NB, H), lambda i: (i, 0)),
            pl.BlockSpec((NB, H), lambda i: (i, 0)),
            pl.BlockSpec((NB, 2 * H), lambda i: (i, 0)),
            pl.BlockSpec((NB, H), lambda i: (i, 0)),
        ],
        out_shape=(
            jax.ShapeDtypeStruct((N, H), f32),
            jax.ShapeDtypeStruct((N, H), f32),
            jax.ShapeDtypeStruct((N, 2 * H), f32),
            jax.ShapeDtypeStruct((N, H), f32),
        ),
    )(h_prev, ahp, nd, wall, ball)


def _edge_math(e, gdb, ge, wc, bc, s3_ref):
    gd = gdb[:, :H]
    gb = gdb[:, H:]
    ehat = jnp.dot(e, wc, preferred_element_type=jnp.float32) + bc + gd + ge
    sig = 1.0 / (1.0 + jnp.exp(-ehat))
    sb = sig * gb
    s3_ref[0] = jnp.concatenate([sb[:, :HH], sig[:, :HH]], axis=1)
    s3_ref[1] = jnp.concatenate([sb[:, HH:], sig[:, HH:]], axis=1)
    return ehat


def _edge1_body(ea_ref, gdb_ref, ge_ref, wine_ref, bine_ref, wc_ref, bc_ref,
                enew_ref, s3_ref):
    e = jnp.dot(ea_ref[...], wine_ref[...],
                preferred_element_type=jnp.float32) + bine_ref[...]
    ehat = _edge_math(e, gdb_ref[...], ge_ref[...], wc_ref[...], bc_ref[...],
                      s3_ref)
    enew_ref[...] = e + jnp.maximum(ehat, 0.0)


def _edge1(ea, gdb, ge, wine, bine, wc, bc):
    f32 = jnp.float32
    d_edge = ea.shape[1]
    return pl.pallas_call(
        _edge1_body,
        grid=(E // EB,),
        in_specs=[
            pl.BlockSpec((EB, d_edge), lambda i: (i, 0)),
            pl.BlockSpec((EB, 2 * H), lambda i: (i, 0)),
            pl.BlockSpec((EB, H), lambda i: (i, 0)),
            pl.BlockSpec((d_edge, H), lambda i: (0, 0)),
            pl.BlockSpec((1, H), lambda i: (0, 0)),
            pl.BlockSpec((H, H), lambda i: (0, 0)),
            pl.BlockSpec((1, H), lambda i: (0, 0)),
        ],
        out_specs=[
            pl.BlockSpec((EB, H), lambda i: (i, 0)),
            pl.BlockSpec((2, EB, H), lambda i: (0, i, 0)),
        ],
        out_shape=(
            jax.ShapeDtypeStruct((E, H), f32),
            jax.ShapeDtypeStruct((2, E, H), f32),
        ),
    )(ea, gdb, ge, wine, bine, wc, bc)


def _edge2_body(e_ref, gdb_ref, ge_ref, wc_ref, bc_ref, s3_ref):
    _edge_math(e_ref[...], gdb_ref[...], ge_ref[...], wc_ref[...], bc_ref[...],
               s3_ref)


def _edge2(e, gdb, ge, wc, bc):
    f32 = jnp.float32
    return pl.pallas_call(
        _edge2_body,
        grid=(E // EB,),
        in_specs=[
            pl.BlockSpec((EB, H), lambda i: (i, 0)),
            pl.BlockSpec((EB, 2 * H), lambda i: (i, 0)),
            pl.BlockSpec((EB, H), lambda i: (i, 0)),
            pl.BlockSpec((H, H), lambda i: (0, 0)),
            pl.BlockSpec((1, H), lambda i: (0, 0)),
        ],
        out_specs=pl.BlockSpec((2, EB, H), lambda i: (0, i, 0)),
        out_shape=jax.ShapeDtypeStruct((2, E, H), f32),
    )(e, gdb, ge, wc, bc)


def _head_body(h_ref, ahp_ref, nd_ref, w1_ref, b1_ref, w2_ref, b2_ref,
               ma_ref, out_ref):
    h = _update_h(h_ref[...], ahp_ref[...], nd_ref[...])
    t = jnp.maximum(
        jnp.dot(h, w1_ref[...], preferred_element_type=jnp.float32)
        + b1_ref[...], 0.0)
    o = jnp.dot(t, w2_ref[...], preferred_element_type=jnp.float32) + b2_ref[...]
    out_ref[...] = ma_ref[...] * jnp.tanh(o)


def _head(h_prev, ahp, nd, w1, b1, w2, b2, ma):
    return pl.pallas_call(
        _head_body,
        grid=(N // NB,),
        in_specs=[
            pl.BlockSpec((NB, H), lambda i: (i, 0)),
            pl.BlockSpec((NB, H), lambda i: (i, 0)),
            pl.BlockSpec((2, NB, H), lambda i: (0, i, 0)),
            pl.BlockSpec((H, MLP_H), lambda i: (0, 0)),
            pl.BlockSpec((1, MLP_H), lambda i: (0, 0)),
            pl.BlockSpec((MLP_H, ACTION_DIM), lambda i: (0, 0)),
            pl.BlockSpec((1, ACTION_DIM), lambda i: (0, 0)),
            pl.BlockSpec((NB, 1), lambda i: (i, 0)),
        ],
        out_specs=pl.BlockSpec((NB, ACTION_DIM), lambda i: (i, 0)),
        out_shape=jax.ShapeDtypeStruct((N, ACTION_DIM), jnp.float32),
    )(h_prev, ahp, nd, w1, b1, w2, b2, ma)


def _sc_gather(dhb, eh, src2, dst2):
    f32 = jnp.float32

    @functools.partial(
        pl.kernel,
        out_type=(
            jax.ShapeDtypeStruct((E, 2 * H), f32),
            jax.ShapeDtypeStruct((E, H), f32),
        ),
        mesh=_vmesh(),
        scratch_types=[],
    )
    def k(dhb_hbm, eh_hbm, si_hbm, di_hbm, gdb_hbm, ge_hbm):
        def body(si_v, di_v, gdb_v, ge_v):
            pltpu.sync_copy(dhb_hbm.at[si_v.at[0]], gdb_v)
            pltpu.sync_copy(eh_hbm.at[di_v.at[0]], ge_v)

        pltpu.emit_pipeline(
            body,
            grid=(NCH,),
            in_specs=[
                pl.BlockSpec((1, CH), lambda i: (0, i)),
                pl.BlockSpec((1, CH), lambda i: (0, i)),
            ],
            out_specs=[
                pl.BlockSpec((CH, 2 * H), lambda i: (i, 0)),
                pl.BlockSpec((CH, H), lambda i: (i, 0)),
            ],
            core_axis_name=("c", "s"),
            dimension_semantics=(pltpu.PARALLEL,),
        )(si_hbm, di_hbm, gdb_hbm, ge_hbm)

    return k(dhb, eh, src2, dst2)


def _sc_scatter(s3, dst2, zeros_nh):
    f32 = jnp.float32

    @functools.partial(
        pl.kernel,
        out_type=jax.ShapeDtypeStruct((2, N, H), f32),
        mesh=_vmesh(),
        scratch_types=[pltpu.VMEM_SHARED((N, H), f32)],
    )
    def k(s3_hbm, di_hbm, z_hbm, nd_hbm, acc):
        cid = lax.axis_index("c")
        sid = lax.axis_index("s")
        row0 = sid * _ZCHUNK

        @pl.when(sid < _NZ)
        def _():
            pltpu.sync_copy(z_hbm.at[pl.ds(row0, _ZCHUNK)],
                            acc.at[pl.ds(row0, _ZCHUNK)])

        plsc.subcore_barrier()

        def body(s_v, di_v):
            pltpu.sync_copy(s_v.at[0], acc.at[di_v.at[0]], add=True)

        pltpu.emit_pipeline(
            body,
            grid=(NCH,),
            in_specs=[
                pl.BlockSpec((1, CH, H), lambda i: (cid, i, 0)),
                pl.BlockSpec((1, CH), lambda i: (0, i)),
            ],
            out_specs=[],
            core_axis_name=("s",),
            dimension_semantics=(pltpu.PARALLEL,),
        )(s3_hbm, di_hbm)

        plsc.subcore_barrier()

        @pl.when(sid < _NZ)
        def _():
            pltpu.sync_copy(acc.at[pl.ds(row0, _ZCHUNK)],
                            nd_hbm.at[cid, pl.ds(row0, _ZCHUNK)])

    return k(s3, dst2, zeros_nh)


def kernel(x, z, edge_attr, max_action, params, edge_index):
    f32 = jnp.float32
    src2 = edge_index[0].reshape(1, E)
    dst2 = edge_index[1].reshape(1, E)
    hx = jnp.concatenate([x, z], axis=1)
    zeros_nh = jnp.zeros((N, H), f32)

    l1, l2 = params["layers"]

    def pack_w(lp):
        w = jnp.concatenate([lp["A"], lp["D"], lp["B"], lp["E"]], axis=1)
        b = jnp.concatenate([lp["Ab"], lp["Db"], lp["Bb"], lp["Eb"]])
        return w, b.reshape(1, 4 * H)

    wall1, ball1 = pack_w(l1)
    wall2, ball2 = pack_w(l2)

    h0, ah1, db1, eh1 = _node1(hx, params["Win_n"],
                               params["bin_n"].reshape(1, H), wall1, ball1)
    gdb1, ge1 = _sc_gather(db1, eh1, src2, dst2)
    e1, s3_1 = _edge1(edge_attr, gdb1, ge1, params["Win_e"],
                      params["bin_e"].reshape(1, H), l1["C"],
                      l1["Cb"].reshape(1, H))
    nd1 = _sc_scatter(s3_1, dst2, zeros_nh)

    h1, ah2, db2, eh2 = _node2(h0, ah1, nd1, wall2, ball2)
    gdb2, ge2 = _sc_gather(db2, eh2, src2, dst2)
    s3_2 = _edge2(e1, gdb2, ge2, l2["C"], l2["Cb"].reshape(1, H))
    nd2 = _sc_scatter(s3_2, dst2, zeros_nh)

    return _head(h1, ah2, nd2, params["W1"], params["b1"].reshape(1, MLP_H),
                 params["W2"], params["b2"].reshape(1, ACTION_DIM), max_action)


# speedup vs baseline: 4.2056x; 1.0327x over previous
import functools

import jax
import jax.numpy as jnp
from jax import lax
from jax.experimental import pallas as pl
from jax.experimental.pallas import tpu as pltpu
from jax.experimental.pallas import tpu_sc as plsc

N = 10000
E = 320000
H = 128
HH = H // 2
MLP_H = 256
ACTION_DIM = 8

NB = 1000
EB = 2000
CH = 128
NCH = E // CH

def _vmesh():
    return plsc.VectorSubcoreMesh(core_axis_name="c", subcore_axis_name="s")


_NSUB = 16
_ZCHUNK = 1000
_NZ = N // _ZCHUNK


def _node1_body(hx_ref, win_ref, binn_ref, wall_ref, ball_ref,
                h_ref, ah_ref, db_ref, eh_ref):
    h = jnp.dot(hx_ref[...], win_ref[...],
                preferred_element_type=jnp.float32) + binn_ref[...]
    h_ref[...] = h
    allv = jnp.dot(h, wall_ref[...],
                   preferred_element_type=jnp.float32) + ball_ref[...]
    ah_ref[...] = allv[:, :H]
    db_ref[...] = allv[:, H:3 * H]
    eh_ref[...] = allv[:, 3 * H:]


def _node1(hx, win, binn, wall, ball):
    f32 = jnp.float32
    return pl.pallas_call(
        _node1_body,
        grid=(N // NB,),
        in_specs=[
            pl.BlockSpec((NB, H), lambda i: (i, 0)),
            pl.BlockSpec((H, H), lambda i: (0, 0)),
            pl.BlockSpec((1, H), lambda i: (0, 0)),
            pl.BlockSpec((H, 4 * H), lambda i: (0, 0)),
            pl.BlockSpec((1, 4 * H), lambda i: (0, 0)),
        ],
        out_specs=[
            pl.BlockSpec((NB, H), lambda i: (i, 0)),
            pl.BlockSpec((NB, H), lambda i: (i, 0)),
            pl.BlockSpec((NB, 2 * H), lambda i: (i, 0)),
            pl.BlockSpec((NB, H), lambda i: (i, 0)),
        ],
        out_shape=(
            jax.ShapeDtypeStruct((N, H), f32),
            jax.ShapeDtypeStruct((N, H), f32),
            jax.ShapeDtypeStruct((N, 2 * H), f32),
            jax.ShapeDtypeStruct((N, H), f32),
        ),
    )(hx, win, binn, wall, ball)


def _update_h(h_prev, ahp, nd):
    num = jnp.concatenate([nd[0, :, :HH], nd[1, :, :HH]], axis=1)
    den = jnp.concatenate([nd[0, :, HH:], nd[1, :, HH:]], axis=1) + 1e-6
    return h_prev + jnp.maximum(ahp + num / den, 0.0)


def _node2_body(h_ref, ahp_ref, nd_ref, wall_ref, ball_ref,
                h_out_ref, ah_ref, db_ref, eh_ref):
    h = _update_h(h_ref[...], ahp_ref[...], nd_ref[...])
    h_out_ref[...] = h
    allv = jnp.dot(h, wall_ref[...],
                   preferred_element_type=jnp.float32) + ball_ref[...]
    ah_ref[...] = allv[:, :H]
    db_ref[...] = allv[:, H:3 * H]
    eh_ref[...] = allv[:, 3 * H:]


def _node2(h_prev, ahp, nd, wall, ball):
    f32 = jnp.float32
    return pl.pallas_call(
        _node2_body,
        grid=(N // NB,),
        in_specs=[
            pl.BlockSpec((NB, H), lambda i: (i, 0)),
            pl.BlockSpec((NB, H), lambda i: (i, 0)),
            pl.BlockSpec((2, NB, H), lambda i: (0, i, 0)),
            pl.BlockSpec((H, 4 * H), lambda i: (0, 0)),
            pl.BlockSpec((1, 4 * H), lambda i: (0, 0)),
        ],
        out_specs=[
            pl.BlockSpec((NB, H), lambda i: (i, 0)),
            pl.BlockSpec((NB, H), lambda i: (i, 0)),
            pl.BlockSpec((NB, 2 * H), lambda i: (i, 0)),
            pl.BlockSpec((NB, H), lambda i: (i, 0)),
        ],
        out_shape=(
            jax.ShapeDtypeStruct((N, H), f32),
            jax.ShapeDtypeStruct((N, H), f32),
            jax.ShapeDtypeStruct((N, 2 * H), f32),
            jax.ShapeDtypeStruct((N, H), f32),
        ),
    )(h_prev, ahp, nd, wall, ball)


def _edge_math(e, gdb, ge, wc, bc, s3_ref):
    gd = gdb[:, :H]
    gb = gdb[:, H:]
    ehat = jnp.dot(e, wc, preferred_element_type=jnp.float32) + bc + gd + ge
    sig = 1.0 / (1.0 + jnp.exp(-ehat))
    sb = sig * gb
    s3_ref[0] = jnp.concatenate([sb[:, :HH], sig[:, :HH]], axis=1)
    s3_ref[1] = jnp.concatenate([sb[:, HH:], sig[:, HH:]], axis=1)
    return ehat


def _edge1_body(ea_ref, gdb_ref, ge_ref, wine_ref, bine_ref, wc_ref, bc_ref,
                enew_ref, s3_ref):
    e = jnp.dot(ea_ref[...], wine_ref[...],
                preferred_element_type=jnp.float32) + bine_ref[...]
    ehat = _edge_math(e, gdb_ref[...], ge_ref[...], wc_ref[...], bc_ref[...],
                      s3_ref)
    enew_ref[...] = e + jnp.maximum(ehat, 0.0)


def _edge1(ea, gdb, ge, wine, bine, wc, bc):
    f32 = jnp.float32
    d_edge = ea.shape[1]
    return pl.pallas_call(
        _edge1_body,
        grid=(E // EB,),
        in_specs=[
            pl.BlockSpec((EB, d_edge), lambda i: (i, 0)),
            pl.BlockSpec((EB, 2 * H), lambda i: (i, 0)),
            pl.BlockSpec((EB, H), lambda i: (i, 0)),
            pl.BlockSpec((d_edge, H), lambda i: (0, 0)),
            pl.BlockSpec((1, H), lambda i: (0, 0)),
            pl.BlockSpec((H, H), lambda i: (0, 0)),
            pl.BlockSpec((1, H), lambda i: (0, 0)),
        ],
        out_specs=[
            pl.BlockSpec((EB, H), lambda i: (i, 0)),
            pl.BlockSpec((2, EB, H), lambda i: (0, i, 0)),
        ],
        out_shape=(
            jax.ShapeDtypeStruct((E, H), f32),
            jax.ShapeDtypeStruct((2, E, H), f32),
        ),
    )(ea, gdb, ge, wine, bine, wc, bc)


def _edge2_body(e_ref, gdb_ref, ge_ref, wc_ref, bc_ref, s3_ref):
    _edge_math(e_ref[...], gdb_ref[...], ge_ref[...], wc_ref[...], bc_ref[...],
               s3_ref)


def _edge2(e, gdb, ge, wc, bc):
    f32 = jnp.float32
    return pl.pallas_call(
        _edge2_body,
        grid=(E // EB,),
        in_specs=[
            pl.BlockSpec((EB, H), lambda i: (i, 0)),
            pl.BlockSpec((EB, 2 * H), lambda i: (i, 0)),
            pl.BlockSpec((EB, H), lambda i: (i, 0)),
            pl.BlockSpec((H, H), lambda i: (0, 0)),
            pl.BlockSpec((1, H), lambda i: (0, 0)),
        ],
        out_specs=pl.BlockSpec((2, EB, H), lambda i: (0, i, 0)),
        out_shape=jax.ShapeDtypeStruct((2, E, H), f32),
    )(e, gdb, ge, wc, bc)


def _head_body(h_ref, ahp_ref, nd_ref, w1_ref, b1_ref, w2_ref, b2_ref,
               ma_ref, out_ref):
    h = _update_h(h_ref[...], ahp_ref[...], nd_ref[...])
    t = jnp.maximum(
        jnp.dot(h, w1_ref[...], preferred_element_type=jnp.float32)
        + b1_ref[...], 0.0)
    o = jnp.dot(t, w2_ref[...], preferred_element_type=jnp.float32) + b2_ref[...]
    out_ref[...] = ma_ref[...] * jnp.tanh(o)


def _head(h_prev, ahp, nd, w1, b1, w2, b2, ma):
    return pl.pallas_call(
        _head_body,
        grid=(N // NB,),
        in_specs=[
            pl.BlockSpec((NB, H), lambda i: (i, 0)),
            pl.BlockSpec((NB, H), lambda i: (i, 0)),
            pl.BlockSpec((2, NB, H), lambda i: (0, i, 0)),
            pl.BlockSpec((H, MLP_H), lambda i: (0, 0)),
            pl.BlockSpec((1, MLP_H), lambda i: (0, 0)),
            pl.BlockSpec((MLP_H, ACTION_DIM), lambda i: (0, 0)),
            pl.BlockSpec((1, ACTION_DIM), lambda i: (0, 0)),
            pl.BlockSpec((NB, 1), lambda i: (i, 0)),
        ],
        out_specs=pl.BlockSpec((NB, ACTION_DIM), lambda i: (i, 0)),
        out_shape=jax.ShapeDtypeStruct((N, ACTION_DIM), jnp.float32),
    )(h_prev, ahp, nd, w1, b1, w2, b2, ma)


def _sc_gather(dhb, eh, src2, dst2):
    f32 = jnp.float32

    @functools.partial(
        pl.kernel,
        out_type=(
            jax.ShapeDtypeStruct((E, 2 * H), f32),
            jax.ShapeDtypeStruct((E, H), f32),
        ),
        mesh=_vmesh(),
        scratch_types=[pltpu.SemaphoreType.DMA, pltpu.SemaphoreType.DMA],
    )
    def k(dhb_hbm, eh_hbm, si_hbm, di_hbm, gdb_hbm, ge_hbm, sem_a, sem_b):
        def body(si_v, di_v, gdb_v, ge_v):
            cp_a = pltpu.async_copy(dhb_hbm.at[si_v.at[0]], gdb_v, sem_a)
            cp_b = pltpu.async_copy(eh_hbm.at[di_v.at[0]], ge_v, sem_b)
            cp_a.wait()
            cp_b.wait()

        pltpu.emit_pipeline(
            body,
            grid=(NCH,),
            in_specs=[
                pl.BlockSpec((1, CH), lambda i: (0, i)),
                pl.BlockSpec((1, CH), lambda i: (0, i)),
            ],
            out_specs=[
                pl.BlockSpec((CH, 2 * H), lambda i: (i, 0)),
                pl.BlockSpec((CH, H), lambda i: (i, 0)),
            ],
            core_axis_name=("c", "s"),
            dimension_semantics=(pltpu.PARALLEL,),
        )(si_hbm, di_hbm, gdb_hbm, ge_hbm)

    return k(dhb, eh, src2, dst2)


def _sc_scatter(s3, dst2, zeros_nh):
    f32 = jnp.float32

    @functools.partial(
        pl.kernel,
        out_type=jax.ShapeDtypeStruct((2, N, H), f32),
        mesh=_vmesh(),
        scratch_types=[pltpu.VMEM_SHARED((N, H), f32)],
    )
    def k(s3_hbm, di_hbm, z_hbm, nd_hbm, acc):
        cid = lax.axis_index("c")
        sid = lax.axis_index("s")
        row0 = sid * _ZCHUNK

        @pl.when(sid < _NZ)
        def _():
            pltpu.sync_copy(z_hbm.at[pl.ds(row0, _ZCHUNK)],
                            acc.at[pl.ds(row0, _ZCHUNK)])

        plsc.subcore_barrier()

        def body(s_v, di_v):
            pltpu.sync_copy(s_v.at[0], acc.at[di_v.at[0]], add=True)

        pltpu.emit_pipeline(
            body,
            grid=(NCH,),
            in_specs=[
                pl.BlockSpec((1, CH, H), lambda i: (cid, i, 0)),
                pl.BlockSpec((1, CH), lambda i: (0, i)),
            ],
            out_specs=[],
            core_axis_name=("s",),
            dimension_semantics=(pltpu.PARALLEL,),
        )(s3_hbm, di_hbm)

        plsc.subcore_barrier()

        @pl.when(sid < _NZ)
        def _():
            pltpu.sync_copy(acc.at[pl.ds(row0, _ZCHUNK)],
                            nd_hbm.at[cid, pl.ds(row0, _ZCHUNK)])

    return k(s3, dst2, zeros_nh)


def kernel(x, z, edge_attr, max_action, params, edge_index):
    f32 = jnp.float32
    src2 = edge_index[0].reshape(1, E)
    dst2 = edge_index[1].reshape(1, E)
    hx = jnp.concatenate([x, z], axis=1)
    zeros_nh = jnp.zeros((N, H), f32)

    l1, l2 = params["layers"]

    def pack_w(lp):
        w = jnp.concatenate([lp["A"], lp["D"], lp["B"], lp["E"]], axis=1)
        b = jnp.concatenate([lp["Ab"], lp["Db"], lp["Bb"], lp["Eb"]])
        return w, b.reshape(1, 4 * H)

    wall1, ball1 = pack_w(l1)
    wall2, ball2 = pack_w(l2)

    h0, ah1, db1, eh1 = _node1(hx, params["Win_n"],
                               params["bin_n"].reshape(1, H), wall1, ball1)
    gdb1, ge1 = _sc_gather(db1, eh1, src2, dst2)
    e1, s3_1 = _edge1(edge_attr, gdb1, ge1, params["Win_e"],
                      params["bin_e"].reshape(1, H), l1["C"],
                      l1["Cb"].reshape(1, H))
    nd1 = _sc_scatter(s3_1, dst2, zeros_nh)

    h1, ah2, db2, eh2 = _node2(h0, ah1, nd1, wall2, ball2)
    gdb2, ge2 = _sc_gather(db2, eh2, src2, dst2)
    s3_2 = _edge2(e1, gdb2, ge2, l2["C"], l2["Cb"].reshape(1, H))
    nd2 = _sc_scatter(s3_2, dst2, zeros_nh)

    return _head(h1, ah2, nd2, params["W1"], params["b1"].reshape(1, MLP_H),
                 params["W2"], params["b2"].reshape(1, ACTION_DIM), max_action)


# speedup vs baseline: 4.9308x; 1.1724x over previous
import functools

import jax
import jax.numpy as jnp
from jax import lax
from jax.experimental import pallas as pl
from jax.experimental.pallas import tpu as pltpu
from jax.experimental.pallas import tpu_sc as plsc

N = 10000
E = 320000
H = 128
HH = H // 2
MLP_H = 256
ACTION_DIM = 8

NB = 1000
EB = 2000
CH = 128
NCH = E // CH

def _vmesh():
    return plsc.VectorSubcoreMesh(core_axis_name="c", subcore_axis_name="s")


_NSUB = 16
_ZCHUNK = 1000
_NZ = N // _ZCHUNK


def _pack_db(dh, bh):
    bits_d = jax.lax.bitcast_convert_type(
        dh.astype(jnp.bfloat16).astype(jnp.float32), jnp.int32)
    bits_b = jax.lax.bitcast_convert_type(
        bh.astype(jnp.bfloat16).astype(jnp.float32), jnp.int32)
    return jax.lax.shift_right_logical(bits_d, 16) | (bits_b & jnp.int32(-65536))


def _unpack_db(p):
    gd = jax.lax.bitcast_convert_type(p << 16, jnp.float32)
    gb = jax.lax.bitcast_convert_type(p & jnp.int32(-65536), jnp.float32)
    return gd, gb


def _node1_body(hx_ref, win_ref, binn_ref, wall_ref, ball_ref,
                h_ref, ah_ref, db_ref, eh_ref):
    h = jnp.dot(hx_ref[...], win_ref[...],
                preferred_element_type=jnp.float32) + binn_ref[...]
    h_ref[...] = h
    allv = jnp.dot(h, wall_ref[...],
                   preferred_element_type=jnp.float32) + ball_ref[...]
    ah_ref[...] = allv[:, :H]
    db_ref[...] = _pack_db(allv[:, H:2 * H], allv[:, 2 * H:3 * H])
    eh_ref[...] = allv[:, 3 * H:]


def _node1(hx, win, binn, wall, ball):
    f32 = jnp.float32
    return pl.pallas_call(
        _node1_body,
        grid=(N // NB,),
        in_specs=[
            pl.BlockSpec((NB, H), lambda i: (i, 0)),
            pl.BlockSpec((H, H), lambda i: (0, 0)),
            pl.BlockSpec((1, H), lambda i: (0, 0)),
            pl.BlockSpec((H, 4 * H), lambda i: (0, 0)),
            pl.BlockSpec((1, 4 * H), lambda i: (0, 0)),
        ],
        out_specs=[
            pl.BlockSpec((NB, H), lambda i: (i, 0)),
            pl.BlockSpec((NB, H), lambda i: (i, 0)),
            pl.BlockSpec((NB, H), lambda i: (i, 0)),
            pl.BlockSpec((NB, H), lambda i: (i, 0)),
        ],
        out_shape=(
            jax.ShapeDtypeStruct((N, H), f32),
            jax.ShapeDtypeStruct((N, H), f32),
            jax.ShapeDtypeStruct((N, H), jnp.int32),
            jax.ShapeDtypeStruct((N, H), f32),
        ),
    )(hx, win, binn, wall, ball)


def _update_h(h_prev, ahp, nd):
    num = jnp.concatenate([nd[0, :, :HH], nd[1, :, :HH]], axis=1)
    den = jnp.concatenate([nd[0, :, HH:], nd[1, :, HH:]], axis=1) + 1e-6
    return h_prev + jnp.maximum(ahp + num / den, 0.0)


def _node2_body(h_ref, ahp_ref, nd_ref, wall_ref, ball_ref,
                h_out_ref, ah_ref, db_ref, eh_ref):
    h = _update_h(h_ref[...], ahp_ref[...], nd_ref[...])
    h_out_ref[...] = h
    allv = jnp.dot(h, wall_ref[...],
                   preferred_element_type=jnp.float32) + ball_ref[...]
    ah_ref[...] = allv[:, :H]
    db_ref[...] = _pack_db(allv[:, H:2 * H], allv[:, 2 * H:3 * H])
    eh_ref[...] = allv[:, 3 * H:]


def _node2(h_prev, ahp, nd, wall, ball):
    f32 = jnp.float32
    return pl.pallas_call(
        _node2_body,
        grid=(N // NB,),
        in_specs=[
            pl.BlockSpec((NB, H), lambda i: (i, 0)),
            pl.BlockSpec((NB, H), lambda i: (i, 0)),
            pl.BlockSpec((2, NB, H), lambda i: (0, i, 0)),
            pl.BlockSpec((H, 4 * H), lambda i: (0, 0)),
            pl.BlockSpec((1, 4 * H), lambda i: (0, 0)),
        ],
        out_specs=[
            pl.BlockSpec((NB, H), lambda i: (i, 0)),
            pl.BlockSpec((NB, H), lambda i: (i, 0)),
            pl.BlockSpec((NB, H), lambda i: (i, 0)),
            pl.BlockSpec((NB, H), lambda i: (i, 0)),
        ],
        out_shape=(
            jax.ShapeDtypeStruct((N, H), f32),
            jax.ShapeDtypeStruct((N, H), f32),
            jax.ShapeDtypeStruct((N, H), jnp.int32),
            jax.ShapeDtypeStruct((N, H), f32),
        ),
    )(h_prev, ahp, nd, wall, ball)


def _edge_math(e, gdb, ge, wc, bc, s3_ref):
    gd, gb = _unpack_db(gdb)
    ehat = jnp.dot(e, wc, preferred_element_type=jnp.float32) + bc + gd + ge
    sig = 1.0 / (1.0 + jnp.exp(-ehat))
    sb = sig * gb
    s3_ref[0] = jnp.concatenate([sb[:, :HH], sig[:, :HH]], axis=1)
    s3_ref[1] = jnp.concatenate([sb[:, HH:], sig[:, HH:]], axis=1)
    return ehat


def _edge1_body(ea_ref, gdb_ref, ge_ref, wine_ref, bine_ref, wc_ref, bc_ref,
                enew_ref, s3_ref):
    e = jnp.dot(ea_ref[...], wine_ref[...],
                preferred_element_type=jnp.float32) + bine_ref[...]
    ehat = _edge_math(e, gdb_ref[...], ge_ref[...], wc_ref[...], bc_ref[...],
                      s3_ref)
    enew_ref[...] = e + jnp.maximum(ehat, 0.0)


def _edge1(ea, gdb, ge, wine, bine, wc, bc):
    f32 = jnp.float32
    d_edge = ea.shape[1]
    return pl.pallas_call(
        _edge1_body,
        grid=(E // EB,),
        in_specs=[
            pl.BlockSpec((EB, d_edge), lambda i: (i, 0)),
            pl.BlockSpec((EB, H), lambda i: (i, 0)),
            pl.BlockSpec((EB, H), lambda i: (i, 0)),
            pl.BlockSpec((d_edge, H), lambda i: (0, 0)),
            pl.BlockSpec((1, H), lambda i: (0, 0)),
            pl.BlockSpec((H, H), lambda i: (0, 0)),
            pl.BlockSpec((1, H), lambda i: (0, 0)),
        ],
        out_specs=[
            pl.BlockSpec((EB, H), lambda i: (i, 0)),
            pl.BlockSpec((2, EB, H), lambda i: (0, i, 0)),
        ],
        out_shape=(
            jax.ShapeDtypeStruct((E, H), f32),
            jax.ShapeDtypeStruct((2, E, H), f32),
        ),
    )(ea, gdb, ge, wine, bine, wc, bc)


def _edge2_body(e_ref, gdb_ref, ge_ref, wc_ref, bc_ref, s3_ref):
    _edge_math(e_ref[...], gdb_ref[...], ge_ref[...], wc_ref[...], bc_ref[...],
               s3_ref)


def _edge2(e, gdb, ge, wc, bc):
    f32 = jnp.float32
    return pl.pallas_call(
        _edge2_body,
        grid=(E // EB,),
        in_specs=[
            pl.BlockSpec((EB, H), lambda i: (i, 0)),
            pl.BlockSpec((EB, H), lambda i: (i, 0)),
            pl.BlockSpec((EB, H), lambda i: (i, 0)),
            pl.BlockSpec((H, H), lambda i: (0, 0)),
            pl.BlockSpec((1, H), lambda i: (0, 0)),
        ],
        out_specs=pl.BlockSpec((2, EB, H), lambda i: (0, i, 0)),
        out_shape=jax.ShapeDtypeStruct((2, E, H), f32),
    )(e, gdb, ge, wc, bc)


def _head_body(h_ref, ahp_ref, nd_ref, w1_ref, b1_ref, w2_ref, b2_ref,
               ma_ref, out_ref):
    h = _update_h(h_ref[...], ahp_ref[...], nd_ref[...])
    t = jnp.maximum(
        jnp.dot(h, w1_ref[...], preferred_element_type=jnp.float32)
        + b1_ref[...], 0.0)
    o = jnp.dot(t, w2_ref[...], preferred_element_type=jnp.float32) + b2_ref[...]
    out_ref[...] = ma_ref[...] * jnp.tanh(o)


def _head(h_prev, ahp, nd, w1, b1, w2, b2, ma):
    return pl.pallas_call(
        _head_body,
        grid=(N // NB,),
        in_specs=[
            pl.BlockSpec((NB, H), lambda i: (i, 0)),
            pl.BlockSpec((NB, H), lambda i: (i, 0)),
            pl.BlockSpec((2, NB, H), lambda i: (0, i, 0)),
            pl.BlockSpec((H, MLP_H), lambda i: (0, 0)),
            pl.BlockSpec((1, MLP_H), lambda i: (0, 0)),
            pl.BlockSpec((MLP_H, ACTION_DIM), lambda i: (0, 0)),
            pl.BlockSpec((1, ACTION_DIM), lambda i: (0, 0)),
            pl.BlockSpec((NB, 1), lambda i: (i, 0)),
        ],
        out_specs=pl.BlockSpec((NB, ACTION_DIM), lambda i: (i, 0)),
        out_shape=jax.ShapeDtypeStruct((N, ACTION_DIM), jnp.float32),
    )(h_prev, ahp, nd, w1, b1, w2, b2, ma)


def _sc_gather(dhb, eh, src2, dst2):
    f32 = jnp.float32

    @functools.partial(
        pl.kernel,
        out_type=(
            jax.ShapeDtypeStruct((E, H), jnp.int32),
            jax.ShapeDtypeStruct((E, H), f32),
        ),
        mesh=_vmesh(),
        scratch_types=[pltpu.SemaphoreType.DMA, pltpu.SemaphoreType.DMA],
    )
    def k(dhb_hbm, eh_hbm, si_hbm, di_hbm, gdb_hbm, ge_hbm, sem_a, sem_b):
        def body(si_v, di_v, gdb_v, ge_v):
            cp_a = pltpu.async_copy(dhb_hbm.at[si_v.at[0]], gdb_v, sem_a)
            cp_b = pltpu.async_copy(eh_hbm.at[di_v.at[0]], ge_v, sem_b)
            cp_a.wait()
            cp_b.wait()

        pltpu.emit_pipeline(
            body,
            grid=(NCH,),
            in_specs=[
                pl.BlockSpec((1, CH), lambda i: (0, i)),
                pl.BlockSpec((1, CH), lambda i: (0, i)),
            ],
            out_specs=[
                pl.BlockSpec((CH, H), lambda i: (i, 0)),
                pl.BlockSpec((CH, H), lambda i: (i, 0)),
            ],
            core_axis_name=("c", "s"),
            dimension_semantics=(pltpu.PARALLEL,),
        )(si_hbm, di_hbm, gdb_hbm, ge_hbm)

    return k(dhb, eh, src2, dst2)


def _sc_scatter(s3, dst2, zeros_nh):
    f32 = jnp.float32

    @functools.partial(
        pl.kernel,
        out_type=jax.ShapeDtypeStruct((2, N, H), f32),
        mesh=_vmesh(),
        scratch_types=[pltpu.VMEM_SHARED((N, H), f32)],
    )
    def k(s3_hbm, di_hbm, z_hbm, nd_hbm, acc):
        cid = lax.axis_index("c")
        sid = lax.axis_index("s")
        row0 = sid * _ZCHUNK

        @pl.when(sid < _NZ)
        def _():
            pltpu.sync_copy(z_hbm.at[pl.ds(row0, _ZCHUNK)],
                            acc.at[pl.ds(row0, _ZCHUNK)])

        plsc.subcore_barrier()

        def body(s_v, di_v):
            pltpu.sync_copy(s_v.at[0], acc.at[di_v.at[0]], add=True)

        pltpu.emit_pipeline(
            body,
            grid=(NCH,),
            in_specs=[
                pl.BlockSpec((1, CH, H), lambda i: (cid, i, 0)),
                pl.BlockSpec((1, CH), lambda i: (0, i)),
            ],
            out_specs=[],
            core_axis_name=("s",),
            dimension_semantics=(pltpu.PARALLEL,),
        )(s3_hbm, di_hbm)

        plsc.subcore_barrier()

        @pl.when(sid < _NZ)
        def _():
            pltpu.sync_copy(acc.at[pl.ds(row0, _ZCHUNK)],
                            nd_hbm.at[cid, pl.ds(row0, _ZCHUNK)])

    return k(s3, dst2, zeros_nh)


def kernel(x, z, edge_attr, max_action, params, edge_index):
    f32 = jnp.float32
    src2 = edge_index[0].reshape(1, E)
    dst2 = edge_index[1].reshape(1, E)
    hx = jnp.concatenate([x, z], axis=1)
    zeros_nh = jnp.zeros((N, H), f32)

    l1, l2 = params["layers"]

    def pack_w(lp):
        w = jnp.concatenate([lp["A"], lp["D"], lp["B"], lp["E"]], axis=1)
        b = jnp.concatenate([lp["Ab"], lp["Db"], lp["Bb"], lp["Eb"]])
        return w, b.reshape(1, 4 * H)

    wall1, ball1 = pack_w(l1)
    wall2, ball2 = pack_w(l2)

    h0, ah1, db1, eh1 = _node1(hx, params["Win_n"],
                               params["bin_n"].reshape(1, H), wall1, ball1)
    gdb1, ge1 = _sc_gather(db1, eh1, src2, dst2)
    e1, s3_1 = _edge1(edge_attr, gdb1, ge1, params["Win_e"],
                      params["bin_e"].reshape(1, H), l1["C"],
                      l1["Cb"].reshape(1, H))
    nd1 = _sc_scatter(s3_1, dst2, zeros_nh)

    h1, ah2, db2, eh2 = _node2(h0, ah1, nd1, wall2, ball2)
    gdb2, ge2 = _sc_gather(db2, eh2, src2, dst2)
    s3_2 = _edge2(e1, gdb2, ge2, l2["C"], l2["Cb"].reshape(1, H))
    nd2 = _sc_scatter(s3_2, dst2, zeros_nh)

    return _head(h1, ah2, nd2, params["W1"], params["b1"].reshape(1, MLP_H),
                 params["W2"], params["b2"].reshape(1, ACTION_DIM), max_action)


# speedup vs baseline: 5.0243x; 1.0190x over previous
import functools

import jax
import jax.numpy as jnp
from jax import lax
from jax.experimental import pallas as pl
from jax.experimental.pallas import tpu as pltpu
from jax.experimental.pallas import tpu_sc as plsc

N = 10000
E = 320000
H = 128
HH = H // 2
MLP_H = 256
ACTION_DIM = 8

NB = 1000
EB = 2000
CH = 128
NCH = E // CH

def _vmesh():
    return plsc.VectorSubcoreMesh(core_axis_name="c", subcore_axis_name="s")


_NSUB = 16
_ZCHUNK = 1000
_NZ = N // _ZCHUNK


def _pack_db(dh, bh):
    bits_d = jax.lax.bitcast_convert_type(
        dh.astype(jnp.bfloat16).astype(jnp.float32), jnp.int32)
    bits_b = jax.lax.bitcast_convert_type(
        bh.astype(jnp.bfloat16).astype(jnp.float32), jnp.int32)
    return jax.lax.shift_right_logical(bits_d, 16) | (bits_b & jnp.int32(-65536))


def _unpack_db(p):
    gd = jax.lax.bitcast_convert_type(p << 16, jnp.float32)
    gb = jax.lax.bitcast_convert_type(p & jnp.int32(-65536), jnp.float32)
    return gd, gb


def _node1_body(hx_ref, win_ref, binn_ref, wall_ref, ball_ref,
                h_ref, ah_ref, db_ref, eh_ref):
    h = jnp.dot(hx_ref[...], win_ref[...],
                preferred_element_type=jnp.float32) + binn_ref[...]
    h_ref[...] = h
    allv = jnp.dot(h, wall_ref[...],
                   preferred_element_type=jnp.float32) + ball_ref[...]
    ah_ref[...] = allv[:, :H]
    db_ref[...] = _pack_db(allv[:, H:2 * H], allv[:, 2 * H:3 * H])
    eh_ref[...] = allv[:, 3 * H:]


def _node1(hx, win, binn, wall, ball):
    f32 = jnp.float32
    return pl.pallas_call(
        _node1_body,
        grid=(N // NB,),
        in_specs=[
            pl.BlockSpec((NB, H), lambda i: (i, 0)),
            pl.BlockSpec((H, H), lambda i: (0, 0)),
            pl.BlockSpec((1, H), lambda i: (0, 0)),
            pl.BlockSpec((H, 4 * H), lambda i: (0, 0)),
            pl.BlockSpec((1, 4 * H), lambda i: (0, 0)),
        ],
        out_specs=[
            pl.BlockSpec((NB, H), lambda i: (i, 0)),
            pl.BlockSpec((NB, H), lambda i: (i, 0)),
            pl.BlockSpec((NB, H), lambda i: (i, 0)),
            pl.BlockSpec((NB, H), lambda i: (i, 0)),
        ],
        out_shape=(
            jax.ShapeDtypeStruct((N, H), f32),
            jax.ShapeDtypeStruct((N, H), f32),
            jax.ShapeDtypeStruct((N, H), jnp.int32),
            jax.ShapeDtypeStruct((N, H), f32),
        ),
    )(hx, win, binn, wall, ball)


def _update_h(h_prev, ahp, nd):
    num = jnp.concatenate([nd[0, :, :HH], nd[1, :, :HH]], axis=1)
    den = jnp.concatenate([nd[0, :, HH:], nd[1, :, HH:]], axis=1) + 1e-6
    return h_prev + jnp.maximum(ahp + num / den, 0.0)


def _node2_body(h_ref, ahp_ref, nd_ref, wall_ref, ball_ref,
                h_out_ref, ah_ref, db_ref, eh_ref):
    h = _update_h(h_ref[...], ahp_ref[...], nd_ref[...])
    h_out_ref[...] = h
    allv = jnp.dot(h, wall_ref[...],
                   preferred_element_type=jnp.float32) + ball_ref[...]
    ah_ref[...] = allv[:, :H]
    db_ref[...] = _pack_db(allv[:, H:2 * H], allv[:, 2 * H:3 * H])
    eh_ref[...] = allv[:, 3 * H:]


def _node2(h_prev, ahp, nd, wall, ball):
    f32 = jnp.float32
    return pl.pallas_call(
        _node2_body,
        grid=(N // NB,),
        in_specs=[
            pl.BlockSpec((NB, H), lambda i: (i, 0)),
            pl.BlockSpec((NB, H), lambda i: (i, 0)),
            pl.BlockSpec((2, NB, H), lambda i: (0, i, 0)),
            pl.BlockSpec((H, 4 * H), lambda i: (0, 0)),
            pl.BlockSpec((1, 4 * H), lambda i: (0, 0)),
        ],
        out_specs=[
            pl.BlockSpec((NB, H), lambda i: (i, 0)),
            pl.BlockSpec((NB, H), lambda i: (i, 0)),
            pl.BlockSpec((NB, H), lambda i: (i, 0)),
            pl.BlockSpec((NB, H), lambda i: (i, 0)),
        ],
        out_shape=(
            jax.ShapeDtypeStruct((N, H), f32),
            jax.ShapeDtypeStruct((N, H), f32),
            jax.ShapeDtypeStruct((N, H), jnp.int32),
            jax.ShapeDtypeStruct((N, H), f32),
        ),
    )(h_prev, ahp, nd, wall, ball)


def _edge_math(e, gdb, ge, wc, bc, s3_ref):
    gd, gb = _unpack_db(gdb)
    ehat = jnp.dot(e, wc, preferred_element_type=jnp.float32) + bc + gd + ge
    sig = 1.0 / (1.0 + jnp.exp(-ehat))
    sb = sig * gb
    s3_ref[0] = jnp.concatenate([sb[:, :HH], sig[:, :HH]], axis=1)
    s3_ref[1] = jnp.concatenate([sb[:, HH:], sig[:, HH:]], axis=1)
    return ehat


def _edge1_body(ea_ref, gdb_ref, ge_ref, wine_ref, bine_ref, wc_ref, bc_ref,
                enew_ref, s3_ref):
    e = jnp.dot(ea_ref[...], wine_ref[...],
                preferred_element_type=jnp.float32) + bine_ref[...]
    ehat = _edge_math(e, gdb_ref[...], ge_ref[...], wc_ref[...], bc_ref[...],
                      s3_ref)
    enew_ref[...] = (e + jnp.maximum(ehat, 0.0)).astype(jnp.bfloat16)


def _edge1(ea, gdb, ge, wine, bine, wc, bc):
    f32 = jnp.float32
    d_edge = ea.shape[1]
    return pl.pallas_call(
        _edge1_body,
        grid=(E // EB,),
        in_specs=[
            pl.BlockSpec((EB, d_edge), lambda i: (i, 0)),
            pl.BlockSpec((EB, H), lambda i: (i, 0)),
            pl.BlockSpec((EB, H), lambda i: (i, 0)),
            pl.BlockSpec((d_edge, H), lambda i: (0, 0)),
            pl.BlockSpec((1, H), lambda i: (0, 0)),
            pl.BlockSpec((H, H), lambda i: (0, 0)),
            pl.BlockSpec((1, H), lambda i: (0, 0)),
        ],
        out_specs=[
            pl.BlockSpec((EB, H), lambda i: (i, 0)),
            pl.BlockSpec((2, EB, H), lambda i: (0, i, 0)),
        ],
        out_shape=(
            jax.ShapeDtypeStruct((E, H), jnp.bfloat16),
            jax.ShapeDtypeStruct((2, E, H), f32),
        ),
    )(ea, gdb, ge, wine, bine, wc, bc)


def _edge2_body(e_ref, gdb_ref, ge_ref, wc_ref, bc_ref, s3_ref):
    _edge_math(e_ref[...].astype(jnp.float32), gdb_ref[...], ge_ref[...],
               wc_ref[...], bc_ref[...], s3_ref)


def _edge2(e, gdb, ge, wc, bc):
    f32 = jnp.float32
    return pl.pallas_call(
        _edge2_body,
        grid=(E // EB,),
        in_specs=[
            pl.BlockSpec((EB, H), lambda i: (i, 0)),
            pl.BlockSpec((EB, H), lambda i: (i, 0)),
            pl.BlockSpec((EB, H), lambda i: (i, 0)),
            pl.BlockSpec((H, H), lambda i: (0, 0)),
            pl.BlockSpec((1, H), lambda i: (0, 0)),
        ],
        out_specs=pl.BlockSpec((2, EB, H), lambda i: (0, i, 0)),
        out_shape=jax.ShapeDtypeStruct((2, E, H), f32),
    )(e, gdb, ge, wc, bc)


def _head_body(h_ref, ahp_ref, nd_ref, w1_ref, b1_ref, w2_ref, b2_ref,
               ma_ref, out_ref):
    h = _update_h(h_ref[...], ahp_ref[...], nd_ref[...])
    t = jnp.maximum(
        jnp.dot(h, w1_ref[...], preferred_element_type=jnp.float32)
        + b1_ref[...], 0.0)
    o = jnp.dot(t, w2_ref[...], preferred_element_type=jnp.float32) + b2_ref[...]
    out_ref[...] = ma_ref[...] * jnp.tanh(o)


def _head(h_prev, ahp, nd, w1, b1, w2, b2, ma):
    return pl.pallas_call(
        _head_body,
        grid=(N // NB,),
        in_specs=[
            pl.BlockSpec((NB, H), lambda i: (i, 0)),
            pl.BlockSpec((NB, H), lambda i: (i, 0)),
            pl.BlockSpec((2, NB, H), lambda i: (0, i, 0)),
            pl.BlockSpec((H, MLP_H), lambda i: (0, 0)),
            pl.BlockSpec((1, MLP_H), lambda i: (0, 0)),
            pl.BlockSpec((MLP_H, ACTION_DIM), lambda i: (0, 0)),
            pl.BlockSpec((1, ACTION_DIM), lambda i: (0, 0)),
            pl.BlockSpec((NB, 1), lambda i: (i, 0)),
        ],
        out_specs=pl.BlockSpec((NB, ACTION_DIM), lambda i: (i, 0)),
        out_shape=jax.ShapeDtypeStruct((N, ACTION_DIM), jnp.float32),
    )(h_prev, ahp, nd, w1, b1, w2, b2, ma)


def _sc_gather(dhb, eh, src2, dst2):
    f32 = jnp.float32

    @functools.partial(
        pl.kernel,
        out_type=(
            jax.ShapeDtypeStruct((E, H), jnp.int32),
            jax.ShapeDtypeStruct((E, H), f32),
        ),
        mesh=_vmesh(),
        scratch_types=[pltpu.SemaphoreType.DMA, pltpu.SemaphoreType.DMA],
    )
    def k(dhb_hbm, eh_hbm, si_hbm, di_hbm, gdb_hbm, ge_hbm, sem_a, sem_b):
        def body(si_v, di_v, gdb_v, ge_v):
            cp_a = pltpu.async_copy(dhb_hbm.at[si_v.at[0]], gdb_v, sem_a)
            cp_b = pltpu.async_copy(eh_hbm.at[di_v.at[0]], ge_v, sem_b)
            cp_a.wait()
            cp_b.wait()

        pltpu.emit_pipeline(
            body,
            grid=(NCH,),
            in_specs=[
                pl.BlockSpec((1, CH), lambda i: (0, i)),
                pl.BlockSpec((1, CH), lambda i: (0, i)),
            ],
            out_specs=[
                pl.BlockSpec((CH, H), lambda i: (i, 0)),
                pl.BlockSpec((CH, H), lambda i: (i, 0)),
            ],
            core_axis_name=("c", "s"),
            dimension_semantics=(pltpu.PARALLEL,),
        )(si_hbm, di_hbm, gdb_hbm, ge_hbm)

    return k(dhb, eh, src2, dst2)


def _sc_scatter(s3, dst2, zeros_nh):
    f32 = jnp.float32

    @functools.partial(
        pl.kernel,
        out_type=jax.ShapeDtypeStruct((2, N, H), f32),
        mesh=_vmesh(),
        scratch_types=[pltpu.VMEM_SHARED((N, H), f32)],
    )
    def k(s3_hbm, di_hbm, z_hbm, nd_hbm, acc):
        cid = lax.axis_index("c")
        sid = lax.axis_index("s")
        row0 = sid * _ZCHUNK

        @pl.when(sid < _NZ)
        def _():
            pltpu.sync_copy(z_hbm.at[pl.ds(row0, _ZCHUNK)],
                            acc.at[pl.ds(row0, _ZCHUNK)])

        plsc.subcore_barrier()

        def body(s_v, di_v):
            pltpu.sync_copy(s_v.at[0], acc.at[di_v.at[0]], add=True)

        pltpu.emit_pipeline(
            body,
            grid=(NCH,),
            in_specs=[
                pl.BlockSpec((1, CH, H), lambda i: (cid, i, 0)),
                pl.BlockSpec((1, CH), lambda i: (0, i)),
            ],
            out_specs=[],
            core_axis_name=("s",),
            dimension_semantics=(pltpu.PARALLEL,),
        )(s3_hbm, di_hbm)

        plsc.subcore_barrier()

        @pl.when(sid < _NZ)
        def _():
            pltpu.sync_copy(acc.at[pl.ds(row0, _ZCHUNK)],
                            nd_hbm.at[cid, pl.ds(row0, _ZCHUNK)])

    return k(s3, dst2, zeros_nh)


def kernel(x, z, edge_attr, max_action, params, edge_index):
    f32 = jnp.float32
    src2 = edge_index[0].reshape(1, E)
    dst2 = edge_index[1].reshape(1, E)
    hx = jnp.concatenate([x, z], axis=1)
    zeros_nh = jnp.zeros((N, H), f32)

    l1, l2 = params["layers"]

    def pack_w(lp):
        w = jnp.concatenate([lp["A"], lp["D"], lp["B"], lp["E"]], axis=1)
        b = jnp.concatenate([lp["Ab"], lp["Db"], lp["Bb"], lp["Eb"]])
        return w, b.reshape(1, 4 * H)

    wall1, ball1 = pack_w(l1)
    wall2, ball2 = pack_w(l2)

    h0, ah1, db1, eh1 = _node1(hx, params["Win_n"],
                               params["bin_n"].reshape(1, H), wall1, ball1)
    gdb1, ge1 = _sc_gather(db1, eh1, src2, dst2)
    e1, s3_1 = _edge1(edge_attr, gdb1, ge1, params["Win_e"],
                      params["bin_e"].reshape(1, H), l1["C"],
                      l1["Cb"].reshape(1, H))
    nd1 = _sc_scatter(s3_1, dst2, zeros_nh)

    h1, ah2, db2, eh2 = _node2(h0, ah1, nd1, wall2, ball2)
    gdb2, ge2 = _sc_gather(db2, eh2, src2, dst2)
    s3_2 = _edge2(e1, gdb2, ge2, l2["C"], l2["Cb"].reshape(1, H))
    nd2 = _sc_scatter(s3_2, dst2, zeros_nh)

    return _head(h1, ah2, nd2, params["W1"], params["b1"].reshape(1, MLP_H),
                 params["W2"], params["b2"].reshape(1, ACTION_DIM), max_action)
